# Initial kernel scaffold; baseline (speedup 1.0000x reference)
import jax
import jax.numpy as jnp
from jax.experimental import pallas as pl


def kernel(input_ids, position_ids, acid_table, pos_table, gamma, beta):
    raise NotImplementedError("write your pallas kernel here")


# speedup vs baseline: 1.6429x; 1.6429x over previous
import functools

import jax
import jax.numpy as jnp
from jax import lax
from jax.experimental import pallas as pl
from jax.experimental.pallas import tpu as pltpu
from jax.experimental.pallas import tpu_sc as plsc

D = 32
G = 128


def _sc_gather(ids3, pids3, acid_table, pos_table, NW, K):
    N = NW * K * G
    mesh = plsc.VectorSubcoreMesh(core_axis_name="c", subcore_axis_name="s")

    @functools.partial(
        pl.kernel,
        out_type=(
            jax.ShapeDtypeStruct((N, D), jnp.float32),
            jax.ShapeDtypeStruct((N, D), jnp.float32),
        ),
        mesh=mesh,
        compiler_params=pltpu.CompilerParams(use_tc_tiling_on_sc=False),
        scratch_types=[
            pltpu.VMEM((K, G), jnp.int32),
            pltpu.VMEM((K, G), jnp.int32),
            pltpu.VMEM((G, D), jnp.float32),
            pltpu.VMEM((G, D), jnp.float32),
            pltpu.SemaphoreType.DMA,
            pltpu.SemaphoreType.DMA,
        ],
    )
    def k(ids_hbm, pids_hbm, acid_hbm, pos_hbm, aout_hbm, pout_hbm,
          idx_v, pidx_v, arows, prows, sema, semp):
        wid = lax.axis_index("s") * 2 + lax.axis_index("c")
        pltpu.sync_copy(ids_hbm.at[wid], idx_v)
        pltpu.sync_copy(pids_hbm.at[wid], pidx_v)
        base = wid * (K * G)

        def body(j, carry):
            cpa = pltpu.async_copy(acid_hbm.at[idx_v.at[j]], arows, sema)
            cpp = pltpu.async_copy(pos_hbm.at[pidx_v.at[j]], prows, semp)
            cpa.wait()
            cpp.wait()
            pltpu.sync_copy(arows, aout_hbm.at[pl.ds(base + j * G, G)])
            pltpu.sync_copy(prows, pout_hbm.at[pl.ds(base + j * G, G)])
            return carry

        lax.fori_loop(0, K, body, 0)

    return k(ids3, pids3, acid_table, pos_table)


def _tc_ln_kernel(a_ref, p_ref, gamma_ref, beta_ref, o_ref):
    x = a_ref[...] + p_ref[...]
    mean = jnp.mean(x, axis=-1, keepdims=True)
    var = jnp.mean((x - mean) ** 2, axis=-1, keepdims=True)
    xhat = (x - mean) * lax.rsqrt(var + 1e-5)
    o_ref[...] = xhat * gamma_ref[...] + beta_ref[...]


def _tc_layernorm(a, p, gamma, beta):
    N = a.shape[0]
    BLK = 8192
    grid = N // BLK
    return pl.pallas_call(
        _tc_ln_kernel,
        grid=(grid,),
        in_specs=[
            pl.BlockSpec((BLK, D), lambda i: (i, 0)),
            pl.BlockSpec((BLK, D), lambda i: (i, 0)),
            pl.BlockSpec((D,), lambda i: (0,)),
            pl.BlockSpec((D,), lambda i: (0,)),
        ],
        out_specs=pl.BlockSpec((BLK, D), lambda i: (i, 0)),
        out_shape=jax.ShapeDtypeStruct((N, D), jnp.float32),
    )(a, p, gamma, beta)


def kernel(input_ids, position_ids, acid_table, pos_table, gamma, beta):
    B, S = input_ids.shape
    N = B * S
    NW = 32
    K = N // (NW * G)
    ids3 = input_ids.reshape(NW, K, G)
    pids3 = position_ids.reshape(NW, K, G)
    arows, prows = _sc_gather(ids3, pids3, acid_table, pos_table, NW, K)
    out = _tc_layernorm(arows, prows, gamma, beta)
    return out.reshape(B, S, D)
